# Initial kernel scaffold; baseline (speedup 1.0000x reference)
import jax
import jax.numpy as jnp
from jax.experimental import pallas as pl


def kernel(relative_position_bias_table, relative_position_index):
    raise NotImplementedError("write your pallas kernel here")


# speedup vs baseline: 20.9276x; 20.9276x over previous
import dataclasses
import functools

import jax
import jax.numpy as jnp
from jax import lax
from jax.experimental import pallas as pl
from jax.experimental.pallas import tpu as pltpu
from jax.experimental.pallas import tpu_sc as plsc

N = 1569
NUM_REL = 10938
NH = 16
HPC = 8
NVREG = 99
TAIL_OFF = N - 16


def _sc_bias_kernel(table_t, idx):
    mesh = plsc.VectorSubcoreMesh(core_axis_name="c", subcore_axis_name="s")

    row_f32 = pltpu.VMEM((N,), jnp.float32)
    row_i32 = pltpu.VMEM((N,), jnp.int32)

    cp = pltpu.CompilerParams()
    if "needs_layout_passes" in pltpu.CompilerParams.__dataclass_fields__:
        cp = dataclasses.replace(cp, needs_layout_passes=False)

    @functools.partial(
        pl.kernel,
        out_type=jax.ShapeDtypeStruct((NH, N, N), jnp.float32),
        mesh=mesh,
        scratch_types=[
            pltpu.VMEM((HPC, NUM_REL), jnp.float32),
            (row_i32, row_i32),
            (tuple(row_f32 for _ in range(HPC)),
             tuple(row_f32 for _ in range(HPC))),
            (pltpu.SemaphoreType.DMA, pltpu.SemaphoreType.DMA),
            (pltpu.SemaphoreType.DMA, pltpu.SemaphoreType.DMA),
        ],
        compiler_params=cp,
    )
    def kern(tab_hbm, idx_hbm, out_hbm, tab_v, idx_v, out_v, sem_i, sem_o):
        c = lax.axis_index("c")
        s = lax.axis_index("s")
        row_base = 98 * s
        head_base = HPC * c

        pltpu.sync_copy(tab_hbm.at[pl.ds(head_base, HPC)], tab_v)

        pltpu.async_copy(idx_hbm.at[row_base], idx_v[0], sem_i[0])

        hvecs = [jnp.full((16,), hh, dtype=jnp.int32) for hh in range(HPC)]

        def wait_idx(bank):
            pltpu.make_async_copy(idx_hbm.at[0], idx_v[bank],
                                  sem_i[bank]).wait()

        def drain_out(bank):
            for hh in range(HPC):
                pltpu.make_async_copy(out_v[bank][hh], out_hbm.at[0, 0],
                                      sem_o[bank]).wait()

        def gather_row(bank):
            @pl.loop(0, NVREG)
            def _(j):
                off = jnp.minimum(16 * j, TAIL_OFF)
                idxv = idx_v[bank][pl.ds(off, 16)]
                for hh in range(HPC):
                    vals = plsc.load_gather(tab_v, [hvecs[hh], idxv])
                    out_v[bank][hh][pl.ds(off, 16)] = vals

        def fire_out(bank, r):
            for hh in range(HPC):
                pltpu.async_copy(out_v[bank][hh],
                                 out_hbm.at[head_base + hh, r], sem_o[bank])

        @pl.loop(0, 100, step=2)
        def _(i):
            r0 = jnp.minimum(row_base + i, N - 1)
            wait_idx(0)
            pltpu.async_copy(idx_hbm.at[jnp.minimum(row_base + i + 1, N - 1)],
                             idx_v[1], sem_i[1])

            @pl.when(i > 0)
            def _():
                drain_out(0)

            gather_row(0)
            fire_out(0, r0)

            r1 = jnp.minimum(row_base + i + 1, N - 1)
            wait_idx(1)

            @pl.when(i < 98)
            def _():
                pltpu.async_copy(
                    idx_hbm.at[jnp.minimum(row_base + i + 2, N - 1)],
                    idx_v[0], sem_i[0])

            @pl.when(i > 0)
            def _():
                drain_out(1)

            gather_row(1)
            fire_out(1, r1)

        drain_out(0)
        drain_out(1)

    return kern(table_t, idx)


def kernel(relative_position_bias_table, relative_position_index):
    table_t = relative_position_bias_table.T
    idx = relative_position_index.astype(jnp.int32)
    return _sc_bias_kernel(table_t, idx)
